# baseline (device time: 34252 ns/iter reference)
import jax
import jax.numpy as jnp
from jax import lax
from jax.experimental import pallas as pl
from jax.experimental.pallas import tpu as pltpu

N_CHUNKS = 8


def kernel(x):
    m, n = x.shape
    n_half = n // 2
    cm = m // N_CHUNKS

    def body(x_ref, out_ref, in_stage, send_buf, recv_buf, out_stage,
             in_sems, send_sems, recv_sems, out_sems, local_sem):
        my_x = lax.axis_index("x")
        my_y = lax.axis_index("y")
        my_z = lax.axis_index("z")
        peer = (1 - my_x, my_y, my_z)
        src_col = (1 - my_x) * n_half
        peer_row0 = (1 - my_x) * m

        barrier_sem = pltpu.get_barrier_semaphore()
        pl.semaphore_signal(
            barrier_sem, inc=1, device_id=peer,
            device_id_type=pl.DeviceIdType.MESH,
        )
        pl.semaphore_wait(barrier_sem, 1)

        local = pltpu.make_async_copy(
            x_ref.at[:, pl.ds(my_x * n_half, n_half)],
            out_ref.at[pl.ds(my_x * m, m), :],
            local_sem,
        )
        local.start()

        def in_dma(i):
            return pltpu.make_async_copy(
                x_ref.at[pl.ds(i * cm, cm), pl.ds(src_col, n_half)],
                in_stage.at[i % 2],
                in_sems.at[i % 2],
            )

        in_dma(0).start()
        rdmas = []
        for i in range(N_CHUNKS):
            slot = i % 2
            if i + 1 < N_CHUNKS:
                in_dma(i + 1).start()
            in_dma(i).wait()
            if i >= 2:
                rdmas[i - 2].wait_send()
            send_buf[slot] = in_stage[slot].astype(jnp.bfloat16)
            rdma = pltpu.make_async_remote_copy(
                src_ref=send_buf.at[slot],
                dst_ref=recv_buf.at[i],
                send_sem=send_sems.at[slot],
                recv_sem=recv_sems.at[i],
                device_id=peer,
                device_id_type=pl.DeviceIdType.MESH,
            )
            rdma.start()
            rdmas.append(rdma)

        out_dmas = []
        for i in range(N_CHUNKS):
            oslot = i % 2
            if i >= 2:
                out_dmas[i - 2].wait()
            rdmas[i].wait_recv()
            out_stage[oslot] = recv_buf[i].astype(jnp.float32)
            od = pltpu.make_async_copy(
                out_stage.at[oslot],
                out_ref.at[pl.ds(peer_row0 + i * cm, cm), :],
                out_sems.at[oslot],
            )
            od.start()
            out_dmas.append(od)

        out_dmas[N_CHUNKS - 2].wait()
        out_dmas[N_CHUNKS - 1].wait()
        rdmas[N_CHUNKS - 2].wait_send()
        rdmas[N_CHUNKS - 1].wait_send()
        local.wait()

    return pl.pallas_call(
        body,
        out_shape=jax.ShapeDtypeStruct((2 * m, n_half), x.dtype),
        in_specs=[pl.BlockSpec(memory_space=pl.ANY)],
        out_specs=pl.BlockSpec(memory_space=pl.ANY),
        scratch_shapes=[
            pltpu.VMEM((2, cm, n_half), x.dtype),
            pltpu.VMEM((2, cm, n_half), jnp.bfloat16),
            pltpu.VMEM((N_CHUNKS, cm, n_half), jnp.bfloat16),
            pltpu.VMEM((2, cm, n_half), x.dtype),
            pltpu.SemaphoreType.DMA((2,)),
            pltpu.SemaphoreType.DMA((2,)),
            pltpu.SemaphoreType.DMA((N_CHUNKS,)),
            pltpu.SemaphoreType.DMA((2,)),
            pltpu.SemaphoreType.DMA,
        ],
        compiler_params=pltpu.CompilerParams(collective_id=0),
    )(x)


# device time: 33253 ns/iter; 1.0300x vs baseline; 1.0300x over previous
import jax
import jax.numpy as jnp
from jax import lax
from jax.experimental import pallas as pl
from jax.experimental.pallas import tpu as pltpu

N_CHUNKS = 8


def kernel(x):
    m, n = x.shape
    n_half = n // 2
    cm = m // N_CHUNKS

    def body(x_ref, out_ref, send_buf, recv_buf, send_sems, recv_sems):
        my_x = lax.axis_index("x")
        my_y = lax.axis_index("y")
        my_z = lax.axis_index("z")
        peer = (1 - my_x, my_y, my_z)
        src_col = (1 - my_x) * n_half

        barrier_sem = pltpu.get_barrier_semaphore()
        pl.semaphore_signal(
            barrier_sem, inc=1, device_id=peer,
            device_id_type=pl.DeviceIdType.MESH,
        )
        send_buf[0] = x_ref[pl.ds(0, cm), pl.ds(src_col, n_half)].astype(
            jnp.bfloat16
        )
        pl.semaphore_wait(barrier_sem, 1)

        rdmas = []
        for i in range(N_CHUNKS):
            slot = i % 2
            if i >= 2:
                rdmas[i - 2].wait_send()
            if i > 0:
                chunk = x_ref[pl.ds(i * cm, cm), pl.ds(src_col, n_half)]
                send_buf[slot] = chunk.astype(jnp.bfloat16)
            rdma = pltpu.make_async_remote_copy(
                src_ref=send_buf.at[slot],
                dst_ref=recv_buf.at[i],
                send_sem=send_sems.at[slot],
                recv_sem=recv_sems.at[i],
                device_id=peer,
                device_id_type=pl.DeviceIdType.MESH,
            )
            rdma.start()
            rdmas.append(rdma)

        out_ref[pl.ds(my_x * m, m), :] = x_ref[:, pl.ds(my_x * n_half, n_half)]

        peer_row0 = (1 - my_x) * m
        for i in range(N_CHUNKS):
            rdmas[i].wait_recv()
            out_ref[pl.ds(peer_row0 + i * cm, cm), :] = recv_buf[i].astype(
                jnp.float32
            )

        rdmas[N_CHUNKS - 2].wait_send()
        rdmas[N_CHUNKS - 1].wait_send()

    return pl.pallas_call(
        body,
        out_shape=jax.ShapeDtypeStruct((2 * m, n_half), x.dtype),
        in_specs=[pl.BlockSpec(memory_space=pltpu.VMEM)],
        out_specs=pl.BlockSpec(memory_space=pltpu.VMEM),
        scratch_shapes=[
            pltpu.VMEM((2, cm, n_half), jnp.bfloat16),
            pltpu.VMEM((N_CHUNKS, cm, n_half), jnp.bfloat16),
            pltpu.SemaphoreType.DMA((2,)),
            pltpu.SemaphoreType.DMA((N_CHUNKS,)),
        ],
        compiler_params=pltpu.CompilerParams(collective_id=0),
    )(x)
